# Initial kernel scaffold; baseline (speedup 1.0000x reference)
import jax
import jax.numpy as jnp
from jax.experimental import pallas as pl


def kernel(edge_index, features, W1, b1, W2, b2):
    raise NotImplementedError("write your pallas kernel here")


# speedup vs baseline: 2.1260x; 2.1260x over previous
import functools

import jax
import jax.numpy as jnp
from jax import lax
from jax.experimental import pallas as pl
from jax.experimental.pallas import tpu as pltpu
from jax.experimental.pallas import tpu_sc as plsc

N = 10000
E = 320000
D = 128
H1 = 16
H2 = 7

NC = 2
NS = 16
NW = NC * NS
NPW = 320
NPAD = NW * NPW

CH = 8000
NCHUNK = E // CH
G = 128


def _make_agg(FD):
    NV = FD // 16
    mesh = plsc.VectorSubcoreMesh(core_axis_name="c", subcore_axis_name="s")

    @functools.partial(
        pl.kernel,
        out_type=jax.ShapeDtypeStruct((NPAD, FD), jnp.float32),
        mesh=mesh,
        compiler_params=pltpu.CompilerParams(
            needs_layout_passes=False,
            use_tc_tiling_on_sc=(FD % 128 == 0),
        ),
        scratch_types=[
            pltpu.VMEM((CH,), jnp.int32),
            pltpu.VMEM((CH,), jnp.int32),
            pltpu.VMEM((CH + 16,), jnp.int32),
            pltpu.VMEM((CH + 16,), jnp.int32),
            pltpu.VMEM((G, FD), jnp.float32),
            pltpu.VMEM((NPW + 1, FD), jnp.float32),
            pltpu.SemaphoreType.DMA,
        ],
    )
    def agg(src_hbm, dst_hbm, feat_hbm, out_hbm,
            dst_v, src_v, cld_v, csrc_v, rows_v, acc_v, sem):
        wid = lax.axis_index("s") * NC + lax.axis_index("c")
        lo = wid * NPW

        neg_inf = jnp.full((16,), -jnp.inf, dtype=jnp.float32)

        def init_acc(r, _):
            for j in range(NV):
                acc_v[r, pl.ds(j * 16, 16)] = neg_inf
            return 0
        lax.fori_loop(0, NPW + 1, init_acc, 0)

        lane = lax.iota(jnp.int32, 16)

        def init_idx(i, _):
            csrc_v[pl.ds(i * 16, 16)] = ((i * 16 + lane) * 31) & 8191
            return 0
        lax.fori_loop(0, (CH + 16) // 16, init_idx, 0)

        def chunk_body(c, _):
            ebase = pl.multiple_of(c * CH, 8)
            pltpu.sync_copy(dst_hbm.at[pl.ds(ebase, CH)], dst_v)
            pltpu.sync_copy(src_hbm.at[pl.ds(ebase, CH)], src_v)

            def filt(g, cnt):
                d16 = dst_v[pl.ds(g * 16, 16)]
                s16 = src_v[pl.ds(g * 16, 16)]
                m = (d16 >= lo) & (d16 < lo + NPW)
                pos = cnt + plsc.cumsum(m.astype(jnp.int32)) - 1
                plsc.store_scatter(cld_v, [pos], d16 - lo, mask=m)
                plsc.store_scatter(csrc_v, [pos], s16, mask=m)
                return cnt + plsc.all_reduce_population_count(m)
            cnt_v = lax.fori_loop(0, CH // 16, filt, jnp.zeros((16,), jnp.int32))
            cnt = jnp.max(cnt_v)

            def batch_body(b, _):
                bbase = pl.multiple_of(b * G, 8)
                pltpu.async_copy(
                    feat_hbm.at[csrc_v.at[pl.ds(bbase, G)]], rows_v, sem
                ).wait()

                def group_body(g, _):
                    ld_vec = cld_v[pl.ds(bbase + g * 16, 16)]
                    for j in range(16):
                        valid = (b * G + g * 16 + j) < cnt
                        ld = jnp.where(valid, ld_vec[j], NPW)
                        e = g * 16 + j
                        for k in range(NV):
                            sl = pl.ds(k * 16, 16)
                            acc_v[ld, sl] = jnp.maximum(acc_v[ld, sl], rows_v[e, sl])
                    return 0
                lax.fori_loop(0, G // 16, group_body, 0)
                return 0
            lax.fori_loop(0, (cnt + G - 1) >> 7, batch_body, 0)
            return 0
        lax.fori_loop(0, NCHUNK, chunk_body, 0)

        pltpu.sync_copy(acc_v.at[pl.ds(0, NPW)], out_hbm.at[pl.ds(lo, NPW)])

    return agg


_agg128 = _make_agg(D)
_agg16 = _make_agg(H1)


def _dense1(x, agg, W, b):
    BM = 1000

    def body(x_ref, a_ref, w_ref, b_ref, o_ref):
        a = a_ref[...]
        a = jnp.where(a == -jnp.inf, 0.0, a)
        r = x_ref[...] + a
        h = jnp.dot(r, w_ref[...], preferred_element_type=jnp.float32) + b_ref[...]
        o_ref[...] = jnp.maximum(h, 0.0)

    return pl.pallas_call(
        body,
        grid=(N // BM,),
        in_specs=[
            pl.BlockSpec((BM, D), lambda i: (i, 0)),
            pl.BlockSpec((BM, D), lambda i: (i, 0)),
            pl.BlockSpec((D, H1), lambda i: (0, 0)),
            pl.BlockSpec((1, H1), lambda i: (0, 0)),
        ],
        out_specs=pl.BlockSpec((BM, H1), lambda i: (i, 0)),
        out_shape=jax.ShapeDtypeStruct((N, H1), jnp.float32),
    )(x, agg, W, b)


def _dense2(h, agg, W, b):
    BM = 1000

    def body(h_ref, a_ref, w_ref, b_ref, o_ref):
        a = a_ref[...]
        a = jnp.where(a == -jnp.inf, 0.0, a)
        r = h_ref[...] + a
        y = jnp.dot(r, w_ref[...], preferred_element_type=jnp.float32) + b_ref[...]
        m = jnp.max(y, axis=-1, keepdims=True)
        z = y - m
        o_ref[...] = z - jnp.log(jnp.sum(jnp.exp(z), axis=-1, keepdims=True))

    return pl.pallas_call(
        body,
        grid=(N // BM,),
        in_specs=[
            pl.BlockSpec((BM, H1), lambda i: (i, 0)),
            pl.BlockSpec((BM, H1), lambda i: (i, 0)),
            pl.BlockSpec((H1, H2), lambda i: (0, 0)),
            pl.BlockSpec((1, H2), lambda i: (0, 0)),
        ],
        out_specs=pl.BlockSpec((BM, H2), lambda i: (i, 0)),
        out_shape=jax.ShapeDtypeStruct((N, H2), jnp.float32),
    )(h, agg, W, b)


def kernel(edge_index, features, W1, b1, W2, b2):
    src = edge_index[0]
    dst = edge_index[1]
    agg1 = _agg128(src, dst, features)
    h = _dense1(features, agg1[:N], W1, b1.reshape(1, H1))
    agg2 = _agg16(src, dst, h)
    return _dense2(h, agg2[:N], W2, b2.reshape(1, H2))


# speedup vs baseline: 3.0872x; 1.4521x over previous
import functools

import jax
import jax.numpy as jnp
from jax import lax
from jax.experimental import pallas as pl
from jax.experimental.pallas import tpu as pltpu
from jax.experimental.pallas import tpu_sc as plsc

N = 10000
E = 320000
D = 128
H1 = 16
H2 = 7

NC = 2
NS = 16
NW = NC * NS
NPW = 320
NPAD = NW * NPW

CH = 6400
NCHUNK = E // CH
G = 64
LMAX = E + NCHUNK * G

_SC_PARAMS = dict(
    needs_layout_passes=False,
)


def _mesh():
    return plsc.VectorSubcoreMesh(core_axis_name="c", subcore_axis_name="s")


def _agg1_build():
    FD = D
    NV = FD // 16

    @functools.partial(
        pl.kernel,
        out_type=(
            jax.ShapeDtypeStruct((NPAD, FD), jnp.float32),
            jax.ShapeDtypeStruct((NW * LMAX,), jnp.int32),
            jax.ShapeDtypeStruct((NW * LMAX,), jnp.int32),
            jax.ShapeDtypeStruct((NW * 16,), jnp.int32),
        ),
        mesh=_mesh(),
        compiler_params=pltpu.CompilerParams(**_SC_PARAMS),
        scratch_types=[
            pltpu.VMEM((CH,), jnp.int32),
            pltpu.VMEM((CH,), jnp.int32),
            pltpu.VMEM((CH + 16,), jnp.int32),
            pltpu.VMEM((CH + 16,), jnp.int32),
            pltpu.VMEM((2, G, FD), jnp.float32),
            pltpu.VMEM((NPW + 1, FD), jnp.float32),
            pltpu.VMEM((16,), jnp.int32),
            pltpu.SemaphoreType.DMA,
            pltpu.SemaphoreType.DMA,
            pltpu.SemaphoreType.DMA,
        ],
    )
    def agg(src_hbm, dst_hbm, feat_hbm,
            out_hbm, llds_hbm, lsrc_hbm, lcnt_hbm,
            dst_v, src_v, cld_v, csrc_v, rows_v, acc_v, cnt_v, sem0, sem1, sem_st):
        wid = lax.axis_index("s") * NC + lax.axis_index("c")
        lo = wid * NPW

        neg_inf = jnp.full((16,), -jnp.inf, dtype=jnp.float32)

        def init_acc(r, _):
            for j in range(NV):
                acc_v[r, pl.ds(j * 16, 16)] = neg_inf
            return 0
        lax.fori_loop(0, NPW + 1, init_acc, 0)

        lane = lax.iota(jnp.int32, 16)
        sems = [sem0, sem1]

        def fire_gather(b, buf, total):
            bbase = pl.multiple_of(b * G, 8)
            pltpu.async_copy(
                feat_hbm.at[csrc_v.at[pl.ds(bbase, G)]],
                rows_v.at[buf], sems[buf])
            pltpu.async_copy(
                cld_v.at[pl.ds(bbase, G)],
                llds_hbm.at[pl.ds(pl.multiple_of(wid * LMAX + total + bbase, 8), G)], sem_st)
            pltpu.async_copy(
                csrc_v.at[pl.ds(bbase, G)],
                lsrc_hbm.at[pl.ds(pl.multiple_of(wid * LMAX + total + bbase, 8), G)], sem_st)

        def wait_gather(buf):
            pltpu.make_async_copy(
                feat_hbm.at[csrc_v.at[pl.ds(0, G)]], rows_v.at[buf], sems[buf]
            ).wait()

        def chunk_body(c, total):
            ebase = pl.multiple_of(c * CH, 8)
            pltpu.sync_copy(dst_hbm.at[pl.ds(ebase, CH)], dst_v)
            pltpu.sync_copy(src_hbm.at[pl.ds(ebase, CH)], src_v)

            def filt(g, cnt):
                d16 = dst_v[pl.ds(g * 16, 16)]
                s16 = src_v[pl.ds(g * 16, 16)]
                m = (d16 >= lo) & (d16 < lo + NPW)
                pos = cnt + plsc.cumsum(m.astype(jnp.int32)) - 1
                plsc.store_scatter(cld_v, [pos], d16 - lo, mask=m)
                plsc.store_scatter(csrc_v, [pos], s16, mask=m)
                return cnt + plsc.all_reduce_population_count(m)
            cnt_vec = lax.fori_loop(0, CH // 16, filt, jnp.zeros((16,), jnp.int32))
            cnt = jnp.max(cnt_vec)
            nb = (cnt + G - 1) >> 6

            def sanit(g, _):
                p16 = g * 16 + lane
                mv = p16 < cnt
                old_ld = cld_v[pl.ds(g * 16, 16)]
                old_sr = csrc_v[pl.ds(g * 16, 16)]
                cld_v[pl.ds(g * 16, 16)] = jnp.where(mv, old_ld, NPW)
                csrc_v[pl.ds(g * 16, 16)] = jnp.where(mv, old_sr, (p16 * 31) & 8191)
                return 0
            lax.fori_loop(cnt >> 4, (nb * G) >> 4, sanit, 0)

            @pl.when(nb > 0)
            def _():
                fire_gather(0, 0, total)

            def do_batch(b, buf):
                wait_gather(buf)

                def group_body(g, _):
                    ld_vec = cld_v[pl.ds(b * G + g * 16, 16)]
                    for j in range(16):
                        ld = ld_vec[j]
                        e = g * 16 + j
                        for k in range(NV):
                            sl = pl.ds(k * 16, 16)
                            r = rows_v[buf, e, sl]
                            acc_v[ld, sl] = jnp.maximum(acc_v[ld, sl], r)
                    return 0
                lax.fori_loop(0, G // 16, group_body, 0)

            def pair_body(p, _):
                b0 = 2 * p
                b1 = b0 + 1

                @pl.when(b1 < nb)
                def _():
                    fire_gather(b1, 1, total)
                do_batch(b0, 0)

                @pl.when(b1 < nb)
                def _():
                    @pl.when(b1 + 1 < nb)
                    def _():
                        fire_gather(b1 + 1, 0, total)
                    do_batch(b1, 1)
                return 0
            lax.fori_loop(0, (nb + 1) >> 1, pair_body, 0)

            def drain(b, _):
                bb = pl.multiple_of(b * G, 8)
                pltpu.make_async_copy(
                    cld_v.at[pl.ds(bb, G)],
                    llds_hbm.at[pl.ds(pl.multiple_of(wid * LMAX + total + bb, 8), G)], sem_st).wait()
                pltpu.make_async_copy(
                    csrc_v.at[pl.ds(bb, G)],
                    lsrc_hbm.at[pl.ds(pl.multiple_of(wid * LMAX + total + bb, 8), G)], sem_st).wait()
                return 0
            lax.fori_loop(0, nb, drain, 0)
            return total + nb * G
        total = lax.fori_loop(0, NCHUNK, chunk_body, 0)

        cnt_v[pl.ds(0, 16)] = jnp.zeros((16,), jnp.int32) + total
        pltpu.sync_copy(cnt_v, lcnt_hbm.at[pl.ds(pl.multiple_of(wid * 16, 8), 16)])

        pltpu.sync_copy(acc_v.at[pl.ds(0, NPW)], out_hbm.at[pl.ds(lo, NPW)])

    return agg


def _agg2_build():
    FD = H1
    SP = 1024
    NBSP = SP // G

    @functools.partial(
        pl.kernel,
        out_type=jax.ShapeDtypeStruct((NPAD, FD), jnp.float32),
        mesh=_mesh(),
        compiler_params=pltpu.CompilerParams(
            use_tc_tiling_on_sc=False, **_SC_PARAMS),
        scratch_types=[
            pltpu.VMEM((SP,), jnp.int32),
            pltpu.VMEM((SP,), jnp.int32),
            pltpu.VMEM((2, G, FD), jnp.float32),
            pltpu.VMEM((NPW + 1, FD), jnp.float32),
            pltpu.VMEM((16,), jnp.int32),
            pltpu.SemaphoreType.DMA,
            pltpu.SemaphoreType.DMA,
            pltpu.SemaphoreType.DMA,
        ],
    )
    def agg(llds_hbm, lsrc_hbm, lcnt_hbm, feat_hbm, out_hbm,
            lds_v, src_v, rows_v, acc_v, cnt_v, sem0, sem1, sem_sp):
        wid = lax.axis_index("s") * NC + lax.axis_index("c")
        lo = wid * NPW

        neg_inf = jnp.full((16,), -jnp.inf, dtype=jnp.float32)

        def init_acc(r, _):
            acc_v[r, pl.ds(0, 16)] = neg_inf
            return 0
        lax.fori_loop(0, NPW + 1, init_acc, 0)

        pltpu.sync_copy(lcnt_hbm.at[pl.ds(pl.multiple_of(wid * 16, 8), 16)], cnt_v)
        total = cnt_v[pl.ds(0, 16)][0]
        nsp = (total + SP - 1) >> 10

        sems = [sem0, sem1]

        def span_load(s):
            sbase = pl.multiple_of(s * SP, 8)
            sz = jnp.minimum(total - s * SP, SP)
            del sz
            pltpu.sync_copy(llds_hbm.at[pl.ds(pl.multiple_of(wid * LMAX + sbase, 8), SP)], lds_v)
            pltpu.sync_copy(lsrc_hbm.at[pl.ds(pl.multiple_of(wid * LMAX + sbase, 8), SP)], src_v)

        def fire_gather(b, buf):
            bbase = pl.multiple_of(b * G, 8)
            pltpu.async_copy(
                feat_hbm.at[src_v.at[pl.ds(bbase, G)]], rows_v.at[buf], sems[buf])

        def wait_gather(buf):
            pltpu.make_async_copy(
                feat_hbm.at[src_v.at[pl.ds(0, G)]], rows_v.at[buf], sems[buf]
            ).wait()

        def span_body(s, _):
            span_load(s)
            nb = jnp.minimum(total - s * SP, SP) >> 6

            @pl.when(nb > 0)
            def _():
                fire_gather(0, 0)

            def do_batch(b, buf):
                wait_gather(buf)

                def group_body(g, _):
                    ld_vec = lds_v[pl.ds(b * G + g * 16, 16)]
                    for j in range(16):
                        ld = ld_vec[j]
                        e = g * 16 + j
                        r = rows_v[buf, e, pl.ds(0, 16)]
                        acc_v[ld, pl.ds(0, 16)] = jnp.maximum(
                            acc_v[ld, pl.ds(0, 16)], r)
                    return 0
                lax.fori_loop(0, G // 16, group_body, 0)

            def pair_body(p, _):
                b0 = 2 * p
                b1 = b0 + 1

                @pl.when(b1 < nb)
                def _():
                    fire_gather(b1, 1)
                do_batch(b0, 0)

                @pl.when(b1 < nb)
                def _():
                    @pl.when(b1 + 1 < nb)
                    def _():
                        fire_gather(b1 + 1, 0)
                    do_batch(b1, 1)
                return 0
            lax.fori_loop(0, (nb + 1) >> 1, pair_body, 0)
            return 0
        lax.fori_loop(0, nsp, span_body, 0)

        pltpu.sync_copy(acc_v.at[pl.ds(0, NPW)], out_hbm.at[pl.ds(lo, NPW)])

    return agg


_agg1 = _agg1_build()
_agg2 = _agg2_build()


def _dense1(x, agg3, W, b):
    BM = 1000

    def body(x_ref, a_ref, w_ref, b_ref, o_ref):
        a = a_ref[...]
        a = jnp.where(a == -jnp.inf, 0.0, a)
        r = x_ref[...] + a
        h = jnp.dot(r, w_ref[...], preferred_element_type=jnp.float32) + b_ref[...]
        o_ref[...] = jnp.maximum(h, 0.0)

    return pl.pallas_call(
        body,
        grid=(N // BM,),
        in_specs=[
            pl.BlockSpec((BM, D), lambda i: (i, 0)),
            pl.BlockSpec((BM, D), lambda i: (i, 0)),
            pl.BlockSpec((D, H1), lambda i: (0, 0)),
            pl.BlockSpec((1, H1), lambda i: (0, 0)),
        ],
        out_specs=pl.BlockSpec((BM, H1), lambda i: (i, 0)),
        out_shape=jax.ShapeDtypeStruct((N, H1), jnp.float32),
    )(x, agg3, W, b)


def _dense2(h, agg, W, b):
    BM = 1000

    def body(h_ref, a_ref, w_ref, b_ref, o_ref):
        a = a_ref[...]
        a = jnp.where(a == -jnp.inf, 0.0, a)
        r = h_ref[...] + a
        y = jnp.dot(r, w_ref[...], preferred_element_type=jnp.float32) + b_ref[...]
        m = jnp.max(y, axis=-1, keepdims=True)
        z = y - m
        o_ref[...] = z - jnp.log(jnp.sum(jnp.exp(z), axis=-1, keepdims=True))

    return pl.pallas_call(
        body,
        grid=(N // BM,),
        in_specs=[
            pl.BlockSpec((BM, H1), lambda i: (i, 0)),
            pl.BlockSpec((BM, H1), lambda i: (i, 0)),
            pl.BlockSpec((H1, H2), lambda i: (0, 0)),
            pl.BlockSpec((1, H2), lambda i: (0, 0)),
        ],
        out_specs=pl.BlockSpec((BM, H2), lambda i: (i, 0)),
        out_shape=jax.ShapeDtypeStruct((N, H2), jnp.float32),
    )(h, agg, W, b)


def kernel(edge_index, features, W1, b1, W2, b2):
    src = edge_index[0]
    dst = edge_index[1]
    agg1, llds, lsrc, lcnt = _agg1(src, dst, features)
    h = _dense1(features, agg1[:N], W1, b1.reshape(1, H1))
    agg2 = _agg2(llds, lsrc, lcnt, h)
    return _dense2(h, agg2[:N], W2, b2.reshape(1, H2))


# speedup vs baseline: 4.0420x; 1.3093x over previous
import functools

import jax
import jax.numpy as jnp
from jax import lax
from jax.experimental import pallas as pl
from jax.experimental.pallas import tpu as pltpu
from jax.experimental.pallas import tpu_sc as plsc

N = 10000
E = 320000
D = 128
H1 = 16
H2 = 7

NC = 2
NS = 16
NW = NC * NS
NPW = 320
NPAD = NW * NPW

CH = 6400
NCHUNK = E // CH
G = 64
LMAX = E + NCHUNK * G

_SC_PARAMS = dict(
    needs_layout_passes=False,
)


def _mesh():
    return plsc.VectorSubcoreMesh(core_axis_name="c", subcore_axis_name="s")


def _agg1_build():
    FD = D
    NV = FD // 16

    @functools.partial(
        pl.kernel,
        out_type=(
            jax.ShapeDtypeStruct((NPAD, FD), jnp.float32),
            jax.ShapeDtypeStruct((NW * LMAX,), jnp.int32),
            jax.ShapeDtypeStruct((NW * LMAX,), jnp.int32),
            jax.ShapeDtypeStruct((NW * 16,), jnp.int32),
        ),
        mesh=_mesh(),
        compiler_params=pltpu.CompilerParams(**_SC_PARAMS),
        scratch_types=[
            pltpu.VMEM((CH,), jnp.int32),
            pltpu.VMEM((CH,), jnp.int32),
            pltpu.VMEM((CH + 16,), jnp.int32),
            pltpu.VMEM((CH + 16,), jnp.int32),
            pltpu.VMEM((2, G, FD), jnp.float32),
            pltpu.VMEM((NPW + 1, FD), jnp.float32),
            pltpu.VMEM((16,), jnp.int32),
            pltpu.SemaphoreType.DMA,
            pltpu.SemaphoreType.DMA,
            pltpu.SemaphoreType.DMA,
        ],
    )
    def agg(src_hbm, dst_hbm, feat_hbm,
            out_hbm, llds_hbm, lsrc_hbm, lcnt_hbm,
            dst_v, src_v, cld_v, csrc_v, rows_v, acc_v, cnt_v, sem0, sem1, sem_st):
        wid = lax.axis_index("s") * NC + lax.axis_index("c")
        lo = wid * NPW

        neg_inf = jnp.full((16,), -jnp.inf, dtype=jnp.float32)

        def init_acc(r, _):
            for j in range(NV):
                acc_v[r, pl.ds(j * 16, 16)] = neg_inf
            return 0
        lax.fori_loop(0, NPW + 1, init_acc, 0)

        lane = lax.iota(jnp.int32, 16)
        sems = [sem0, sem1]

        def fire_gather(b, buf, total):
            bbase = pl.multiple_of(b * G, 8)
            pltpu.async_copy(
                feat_hbm.at[csrc_v.at[pl.ds(bbase, G)]],
                rows_v.at[buf], sems[buf])
            pltpu.async_copy(
                cld_v.at[pl.ds(bbase, G)],
                llds_hbm.at[pl.ds(pl.multiple_of(wid * LMAX + total + bbase, 8), G)], sem_st)
            pltpu.async_copy(
                csrc_v.at[pl.ds(bbase, G)],
                lsrc_hbm.at[pl.ds(pl.multiple_of(wid * LMAX + total + bbase, 8), G)], sem_st)

        def wait_gather(buf):
            pltpu.make_async_copy(
                feat_hbm.at[csrc_v.at[pl.ds(0, G)]], rows_v.at[buf], sems[buf]
            ).wait()

        def chunk_body(c, total):
            ebase = pl.multiple_of(c * CH, 8)
            pltpu.sync_copy(dst_hbm.at[pl.ds(ebase, CH)], dst_v)
            pltpu.sync_copy(src_hbm.at[pl.ds(ebase, CH)], src_v)

            def filt(g, cnt):
                d16 = dst_v[pl.ds(g * 16, 16)]
                s16 = src_v[pl.ds(g * 16, 16)]
                m = (d16 >= lo) & (d16 < lo + NPW)
                pos = cnt + plsc.cumsum(m.astype(jnp.int32)) - 1
                plsc.store_scatter(cld_v, [pos], d16 - lo, mask=m)
                plsc.store_scatter(csrc_v, [pos], s16, mask=m)
                return cnt + plsc.all_reduce_population_count(m)
            cnt_vec = lax.fori_loop(0, CH // 16, filt, jnp.zeros((16,), jnp.int32))
            cnt = jnp.max(cnt_vec)
            nb = (cnt + G - 1) >> 6

            def sanit(g, _):
                p16 = g * 16 + lane
                mv = p16 < cnt
                old_ld = cld_v[pl.ds(g * 16, 16)]
                old_sr = csrc_v[pl.ds(g * 16, 16)]
                cld_v[pl.ds(g * 16, 16)] = jnp.where(mv, old_ld, NPW)
                csrc_v[pl.ds(g * 16, 16)] = jnp.where(mv, old_sr, (p16 * 31) & 8191)
                return 0
            lax.fori_loop(cnt >> 4, (nb * G) >> 4, sanit, 0)

            @pl.when(nb > 0)
            def _():
                fire_gather(0, 0, total)

            def do_batch(b, buf):
                wait_gather(buf)

                def group_body(g, _):
                    ld_vec = cld_v[pl.ds(b * G + g * 16, 16)]
                    for j in range(16):
                        ld = ld_vec[j]
                        e = g * 16 + j

                        @plsc.parallel_loop(0, NV, unroll=NV)
                        def _(k):
                            sl = pl.ds(k * 16, 16)
                            r = rows_v[buf, e, sl]
                            acc_v[ld, sl] = jnp.maximum(acc_v[ld, sl], r)
                    return 0
                lax.fori_loop(0, G // 16, group_body, 0)

            def pair_body(p, _):
                b0 = 2 * p
                b1 = b0 + 1

                @pl.when(b1 < nb)
                def _():
                    fire_gather(b1, 1, total)
                do_batch(b0, 0)

                @pl.when(b1 < nb)
                def _():
                    @pl.when(b1 + 1 < nb)
                    def _():
                        fire_gather(b1 + 1, 0, total)
                    do_batch(b1, 1)
                return 0
            lax.fori_loop(0, (nb + 1) >> 1, pair_body, 0)

            def drain(b, _):
                bb = pl.multiple_of(b * G, 8)
                pltpu.make_async_copy(
                    cld_v.at[pl.ds(bb, G)],
                    llds_hbm.at[pl.ds(pl.multiple_of(wid * LMAX + total + bb, 8), G)], sem_st).wait()
                pltpu.make_async_copy(
                    csrc_v.at[pl.ds(bb, G)],
                    lsrc_hbm.at[pl.ds(pl.multiple_of(wid * LMAX + total + bb, 8), G)], sem_st).wait()
                return 0
            lax.fori_loop(0, nb, drain, 0)
            return total + nb * G
        total = lax.fori_loop(0, NCHUNK, chunk_body, 0)

        cnt_v[pl.ds(0, 16)] = jnp.zeros((16,), jnp.int32) + total
        pltpu.sync_copy(cnt_v, lcnt_hbm.at[pl.ds(pl.multiple_of(wid * 16, 8), 16)])

        pltpu.sync_copy(acc_v.at[pl.ds(0, NPW)], out_hbm.at[pl.ds(lo, NPW)])

    return agg


def _agg2_build():
    FD = H1
    SP = 1024
    NBSP = SP // G

    @functools.partial(
        pl.kernel,
        out_type=jax.ShapeDtypeStruct((NPAD, FD), jnp.float32),
        mesh=_mesh(),
        compiler_params=pltpu.CompilerParams(
            use_tc_tiling_on_sc=False, **_SC_PARAMS),
        scratch_types=[
            pltpu.VMEM((SP,), jnp.int32),
            pltpu.VMEM((SP,), jnp.int32),
            pltpu.VMEM((2, G, FD), jnp.float32),
            pltpu.VMEM((NPW + 1, FD), jnp.float32),
            pltpu.VMEM((16,), jnp.int32),
            pltpu.SemaphoreType.DMA,
            pltpu.SemaphoreType.DMA,
            pltpu.SemaphoreType.DMA,
        ],
    )
    def agg(llds_hbm, lsrc_hbm, lcnt_hbm, feat_hbm, out_hbm,
            lds_v, src_v, rows_v, acc_v, cnt_v, sem0, sem1, sem_sp):
        wid = lax.axis_index("s") * NC + lax.axis_index("c")
        lo = wid * NPW

        neg_inf = jnp.full((16,), -jnp.inf, dtype=jnp.float32)

        def init_acc(r, _):
            acc_v[r, pl.ds(0, 16)] = neg_inf
            return 0
        lax.fori_loop(0, NPW + 1, init_acc, 0)

        pltpu.sync_copy(lcnt_hbm.at[pl.ds(pl.multiple_of(wid * 16, 8), 16)], cnt_v)
        total = cnt_v[pl.ds(0, 16)][0]
        nsp = (total + SP - 1) >> 10

        sems = [sem0, sem1]

        def span_load(s):
            sbase = pl.multiple_of(s * SP, 8)
            sz = jnp.minimum(total - s * SP, SP)
            del sz
            pltpu.sync_copy(llds_hbm.at[pl.ds(pl.multiple_of(wid * LMAX + sbase, 8), SP)], lds_v)
            pltpu.sync_copy(lsrc_hbm.at[pl.ds(pl.multiple_of(wid * LMAX + sbase, 8), SP)], src_v)

        def fire_gather(b, buf):
            bbase = pl.multiple_of(b * G, 8)
            pltpu.async_copy(
                feat_hbm.at[src_v.at[pl.ds(bbase, G)]], rows_v.at[buf], sems[buf])

        def wait_gather(buf):
            pltpu.make_async_copy(
                feat_hbm.at[src_v.at[pl.ds(0, G)]], rows_v.at[buf], sems[buf]
            ).wait()

        def span_body(s, _):
            span_load(s)
            nb = jnp.minimum(total - s * SP, SP) >> 6

            @pl.when(nb > 0)
            def _():
                fire_gather(0, 0)

            def do_batch(b, buf):
                wait_gather(buf)

                def group_body(g, _):
                    ld_vec = lds_v[pl.ds(b * G + g * 16, 16)]
                    for j in range(16):
                        ld = ld_vec[j]
                        e = g * 16 + j
                        r = rows_v[buf, e, pl.ds(0, 16)]
                        acc_v[ld, pl.ds(0, 16)] = jnp.maximum(
                            acc_v[ld, pl.ds(0, 16)], r)
                    return 0
                lax.fori_loop(0, G // 16, group_body, 0)

            def pair_body(p, _):
                b0 = 2 * p
                b1 = b0 + 1

                @pl.when(b1 < nb)
                def _():
                    fire_gather(b1, 1)
                do_batch(b0, 0)

                @pl.when(b1 < nb)
                def _():
                    @pl.when(b1 + 1 < nb)
                    def _():
                        fire_gather(b1 + 1, 0)
                    do_batch(b1, 1)
                return 0
            lax.fori_loop(0, (nb + 1) >> 1, pair_body, 0)
            return 0
        lax.fori_loop(0, nsp, span_body, 0)

        pltpu.sync_copy(acc_v.at[pl.ds(0, NPW)], out_hbm.at[pl.ds(lo, NPW)])

    return agg


_agg1 = _agg1_build()
_agg2 = _agg2_build()


def _dense1(x, agg3, W, b):
    BM = 1000

    def body(x_ref, a_ref, w_ref, b_ref, o_ref):
        a = a_ref[...]
        a = jnp.where(a == -jnp.inf, 0.0, a)
        r = x_ref[...] + a
        h = jnp.dot(r, w_ref[...], preferred_element_type=jnp.float32) + b_ref[...]
        o_ref[...] = jnp.maximum(h, 0.0)

    return pl.pallas_call(
        body,
        grid=(N // BM,),
        in_specs=[
            pl.BlockSpec((BM, D), lambda i: (i, 0)),
            pl.BlockSpec((BM, D), lambda i: (i, 0)),
            pl.BlockSpec((D, H1), lambda i: (0, 0)),
            pl.BlockSpec((1, H1), lambda i: (0, 0)),
        ],
        out_specs=pl.BlockSpec((BM, H1), lambda i: (i, 0)),
        out_shape=jax.ShapeDtypeStruct((N, H1), jnp.float32),
    )(x, agg3, W, b)


def _dense2(h, agg, W, b):
    BM = 1000

    def body(h_ref, a_ref, w_ref, b_ref, o_ref):
        a = a_ref[...]
        a = jnp.where(a == -jnp.inf, 0.0, a)
        r = h_ref[...] + a
        y = jnp.dot(r, w_ref[...], preferred_element_type=jnp.float32) + b_ref[...]
        m = jnp.max(y, axis=-1, keepdims=True)
        z = y - m
        o_ref[...] = z - jnp.log(jnp.sum(jnp.exp(z), axis=-1, keepdims=True))

    return pl.pallas_call(
        body,
        grid=(N // BM,),
        in_specs=[
            pl.BlockSpec((BM, H1), lambda i: (i, 0)),
            pl.BlockSpec((BM, H1), lambda i: (i, 0)),
            pl.BlockSpec((H1, H2), lambda i: (0, 0)),
            pl.BlockSpec((1, H2), lambda i: (0, 0)),
        ],
        out_specs=pl.BlockSpec((BM, H2), lambda i: (i, 0)),
        out_shape=jax.ShapeDtypeStruct((N, H2), jnp.float32),
    )(h, agg, W, b)


def kernel(edge_index, features, W1, b1, W2, b2):
    src = edge_index[0]
    dst = edge_index[1]
    agg1, llds, lsrc, lcnt = _agg1(src, dst, features)
    h = _dense1(features, agg1[:N], W1, b1.reshape(1, H1))
    agg2 = _agg2(llds, lsrc, lcnt, h)
    return _dense2(h, agg2[:N], W2, b2.reshape(1, H2))


# speedup vs baseline: 5.5342x; 1.3692x over previous
import functools

import jax
import jax.numpy as jnp
from jax import lax
from jax.experimental import pallas as pl
from jax.experimental.pallas import tpu as pltpu
from jax.experimental.pallas import tpu_sc as plsc

N = 10000
E = 320000
D = 128
H1 = 16
H2 = 7

NC = 2
NS = 16
NW = NC * NS
NPW = 320
NPAD = NW * NPW

CH = 6400
NCHUNK = E // CH
G = 64
LMAX = E + NCHUNK * G

_SC_PARAMS = dict(
    needs_layout_passes=False,
)


def _mesh():
    return plsc.VectorSubcoreMesh(core_axis_name="c", subcore_axis_name="s")


def _agg1_build():
    FD = D
    NV = FD // 16

    @functools.partial(
        pl.kernel,
        out_type=(
            jax.ShapeDtypeStruct((NPAD, FD), jnp.float32),
            jax.ShapeDtypeStruct((NW * LMAX,), jnp.int32),
            jax.ShapeDtypeStruct((NW * LMAX,), jnp.int32),
            jax.ShapeDtypeStruct((NW * 16,), jnp.int32),
        ),
        mesh=_mesh(),
        compiler_params=pltpu.CompilerParams(**_SC_PARAMS),
        scratch_types=[
            pltpu.VMEM((CH,), jnp.int32),
            pltpu.VMEM((CH,), jnp.int32),
            pltpu.VMEM((CH + 16,), jnp.int32),
            pltpu.VMEM((CH + 16,), jnp.int32),
            pltpu.VMEM((2, G, FD), jnp.float32),
            pltpu.VMEM((NPW + 1, FD), jnp.float32),
            pltpu.VMEM((16,), jnp.int32),
            pltpu.SemaphoreType.DMA,
            pltpu.SemaphoreType.DMA,
            pltpu.SemaphoreType.DMA,
        ],
    )
    def agg(src_hbm, dst_hbm, feat_hbm,
            out_hbm, llds_hbm, lsrc_hbm, lcnt_hbm,
            dst_v, src_v, cld_v, csrc_v, rows_v, acc_v, cnt_v, sem0, sem1, sem_st):
        wid = lax.axis_index("s") * NC + lax.axis_index("c")
        lo = wid * NPW

        neg_inf = jnp.full((16,), -jnp.inf, dtype=jnp.float32)

        def init_acc(r, _):
            for j in range(NV):
                acc_v[r, pl.ds(j * 16, 16)] = neg_inf
            return 0
        lax.fori_loop(0, NPW + 1, init_acc, 0)

        lane = lax.iota(jnp.int32, 16)
        sems = [sem0, sem1]

        def fire_gather(b, buf, total):
            bbase = pl.multiple_of(b * G, 8)
            pltpu.async_copy(
                feat_hbm.at[csrc_v.at[pl.ds(bbase, G)]],
                rows_v.at[buf], sems[buf])
            pltpu.async_copy(
                cld_v.at[pl.ds(bbase, G)],
                llds_hbm.at[pl.ds(pl.multiple_of(wid * LMAX + total + bbase, 8), G)], sem_st)
            pltpu.async_copy(
                csrc_v.at[pl.ds(bbase, G)],
                lsrc_hbm.at[pl.ds(pl.multiple_of(wid * LMAX + total + bbase, 8), G)], sem_st)

        def wait_gather(buf):
            pltpu.make_async_copy(
                feat_hbm.at[csrc_v.at[pl.ds(0, G)]], rows_v.at[buf], sems[buf]
            ).wait()

        def chunk_body(c, total):
            ebase = pl.multiple_of(c * CH, 8)
            pltpu.sync_copy(dst_hbm.at[pl.ds(ebase, CH)], dst_v)
            pltpu.sync_copy(src_hbm.at[pl.ds(ebase, CH)], src_v)

            @plsc.parallel_loop(0, CH // 16, unroll=4,
                                carry=jnp.zeros((16,), jnp.int32))
            def filt(g, cnt):
                d16 = dst_v[pl.ds(g * 16, 16)]
                s16 = src_v[pl.ds(g * 16, 16)]
                m = (d16 >= lo) & (d16 < lo + NPW)
                pos = cnt + plsc.cumsum(m.astype(jnp.int32)) - 1
                plsc.store_scatter(cld_v, [pos], d16 - lo, mask=m)
                plsc.store_scatter(csrc_v, [pos], s16, mask=m)
                return cnt + plsc.all_reduce_population_count(m)
            cnt_vec = filt
            cnt = jnp.max(cnt_vec)
            nb = (cnt + G - 1) >> 6

            def sanit(g, _):
                p16 = g * 16 + lane
                mv = p16 < cnt
                old_ld = cld_v[pl.ds(g * 16, 16)]
                old_sr = csrc_v[pl.ds(g * 16, 16)]
                cld_v[pl.ds(g * 16, 16)] = jnp.where(mv, old_ld, NPW)
                csrc_v[pl.ds(g * 16, 16)] = jnp.where(mv, old_sr, (p16 * 31) & 8191)
                return 0
            lax.fori_loop(cnt >> 4, (nb * G) >> 4, sanit, 0)

            @pl.when(nb > 0)
            def _():
                fire_gather(0, 0, total)

            def do_batch(b, buf):
                wait_gather(buf)

                def group_body(g, _):
                    ld_vec = cld_v[pl.ds(b * G + g * 16, 16)]
                    for j in range(16):
                        ld = ld_vec[j]
                        e = g * 16 + j

                        @plsc.parallel_loop(0, NV, unroll=NV)
                        def _(k):
                            sl = pl.ds(k * 16, 16)
                            r = rows_v[buf, e, sl]
                            acc_v[ld, sl] = jnp.maximum(acc_v[ld, sl], r)
                    return 0
                lax.fori_loop(0, G // 16, group_body, 0)

            def pair_body(p, _):
                b0 = 2 * p
                b1 = b0 + 1

                @pl.when(b1 < nb)
                def _():
                    fire_gather(b1, 1, total)
                do_batch(b0, 0)

                @pl.when(b1 < nb)
                def _():
                    @pl.when(b1 + 1 < nb)
                    def _():
                        fire_gather(b1 + 1, 0, total)
                    do_batch(b1, 1)
                return 0
            lax.fori_loop(0, (nb + 1) >> 1, pair_body, 0)

            def drain(b, _):
                bb = pl.multiple_of(b * G, 8)
                pltpu.make_async_copy(
                    cld_v.at[pl.ds(bb, G)],
                    llds_hbm.at[pl.ds(pl.multiple_of(wid * LMAX + total + bb, 8), G)], sem_st).wait()
                pltpu.make_async_copy(
                    csrc_v.at[pl.ds(bb, G)],
                    lsrc_hbm.at[pl.ds(pl.multiple_of(wid * LMAX + total + bb, 8), G)], sem_st).wait()
                return 0
            lax.fori_loop(0, nb, drain, 0)
            return total + nb * G
        total = lax.fori_loop(0, NCHUNK, chunk_body, 0)

        cnt_v[pl.ds(0, 16)] = jnp.zeros((16,), jnp.int32) + total
        pltpu.sync_copy(cnt_v, lcnt_hbm.at[pl.ds(pl.multiple_of(wid * 16, 8), 16)])

        pltpu.sync_copy(acc_v.at[pl.ds(0, NPW)], out_hbm.at[pl.ds(lo, NPW)])

    return agg


def _agg2_build():
    FD = H1
    SP = 1024
    NBSP = SP // G

    @functools.partial(
        pl.kernel,
        out_type=jax.ShapeDtypeStruct((NPAD, FD), jnp.float32),
        mesh=_mesh(),
        compiler_params=pltpu.CompilerParams(
            use_tc_tiling_on_sc=False, **_SC_PARAMS),
        scratch_types=[
            pltpu.VMEM((SP,), jnp.int32),
            pltpu.VMEM((SP,), jnp.int32),
            pltpu.VMEM((2, G, FD), jnp.float32),
            pltpu.VMEM((NPW + 1, FD), jnp.float32),
            pltpu.VMEM((16,), jnp.int32),
            pltpu.SemaphoreType.DMA,
            pltpu.SemaphoreType.DMA,
            pltpu.SemaphoreType.DMA,
        ],
    )
    def agg(llds_hbm, lsrc_hbm, lcnt_hbm, feat_hbm, out_hbm,
            lds_v, src_v, rows_v, acc_v, cnt_v, sem0, sem1, sem_sp):
        wid = lax.axis_index("s") * NC + lax.axis_index("c")
        lo = wid * NPW

        neg_inf = jnp.full((16,), -jnp.inf, dtype=jnp.float32)

        def init_acc(r, _):
            acc_v[r, pl.ds(0, 16)] = neg_inf
            return 0
        lax.fori_loop(0, NPW + 1, init_acc, 0)

        pltpu.sync_copy(lcnt_hbm.at[pl.ds(pl.multiple_of(wid * 16, 8), 16)], cnt_v)
        total = cnt_v[pl.ds(0, 16)][0]
        nsp = (total + SP - 1) >> 10

        sems = [sem0, sem1]

        def span_load(s):
            sbase = pl.multiple_of(s * SP, 8)
            sz = jnp.minimum(total - s * SP, SP)
            del sz
            pltpu.sync_copy(llds_hbm.at[pl.ds(pl.multiple_of(wid * LMAX + sbase, 8), SP)], lds_v)
            pltpu.sync_copy(lsrc_hbm.at[pl.ds(pl.multiple_of(wid * LMAX + sbase, 8), SP)], src_v)

        def fire_gather(b, buf):
            bbase = pl.multiple_of(b * G, 8)
            pltpu.async_copy(
                feat_hbm.at[src_v.at[pl.ds(bbase, G)]], rows_v.at[buf], sems[buf])

        def wait_gather(buf):
            pltpu.make_async_copy(
                feat_hbm.at[src_v.at[pl.ds(0, G)]], rows_v.at[buf], sems[buf]
            ).wait()

        def span_body(s, _):
            span_load(s)
            nb = jnp.minimum(total - s * SP, SP) >> 6

            @pl.when(nb > 0)
            def _():
                fire_gather(0, 0)

            def do_batch(b, buf):
                wait_gather(buf)

                def group_body(g, _):
                    ld_vec = lds_v[pl.ds(b * G + g * 16, 16)]
                    for j in range(16):
                        ld = ld_vec[j]
                        e = g * 16 + j
                        r = rows_v[buf, e, pl.ds(0, 16)]
                        acc_v[ld, pl.ds(0, 16)] = jnp.maximum(
                            acc_v[ld, pl.ds(0, 16)], r)
                    return 0
                lax.fori_loop(0, G // 16, group_body, 0)

            def pair_body(p, _):
                b0 = 2 * p
                b1 = b0 + 1

                @pl.when(b1 < nb)
                def _():
                    fire_gather(b1, 1)
                do_batch(b0, 0)

                @pl.when(b1 < nb)
                def _():
                    @pl.when(b1 + 1 < nb)
                    def _():
                        fire_gather(b1 + 1, 0)
                    do_batch(b1, 1)
                return 0
            lax.fori_loop(0, (nb + 1) >> 1, pair_body, 0)
            return 0
        lax.fori_loop(0, nsp, span_body, 0)

        pltpu.sync_copy(acc_v.at[pl.ds(0, NPW)], out_hbm.at[pl.ds(lo, NPW)])

    return agg


_agg1 = _agg1_build()
_agg2 = _agg2_build()


def _dense1(x, agg3, W, b):
    BM = 1000

    def body(x_ref, a_ref, w_ref, b_ref, o_ref):
        a = a_ref[...]
        a = jnp.where(a == -jnp.inf, 0.0, a)
        r = x_ref[...] + a
        h = jnp.dot(r, w_ref[...], preferred_element_type=jnp.float32) + b_ref[...]
        o_ref[...] = jnp.maximum(h, 0.0)

    return pl.pallas_call(
        body,
        grid=(N // BM,),
        in_specs=[
            pl.BlockSpec((BM, D), lambda i: (i, 0)),
            pl.BlockSpec((BM, D), lambda i: (i, 0)),
            pl.BlockSpec((D, H1), lambda i: (0, 0)),
            pl.BlockSpec((1, H1), lambda i: (0, 0)),
        ],
        out_specs=pl.BlockSpec((BM, H1), lambda i: (i, 0)),
        out_shape=jax.ShapeDtypeStruct((N, H1), jnp.float32),
    )(x, agg3, W, b)


def _dense2(h, agg, W, b):
    BM = 1000

    def body(h_ref, a_ref, w_ref, b_ref, o_ref):
        a = a_ref[...]
        a = jnp.where(a == -jnp.inf, 0.0, a)
        r = h_ref[...] + a
        y = jnp.dot(r, w_ref[...], preferred_element_type=jnp.float32) + b_ref[...]
        m = jnp.max(y, axis=-1, keepdims=True)
        z = y - m
        o_ref[...] = z - jnp.log(jnp.sum(jnp.exp(z), axis=-1, keepdims=True))

    return pl.pallas_call(
        body,
        grid=(N // BM,),
        in_specs=[
            pl.BlockSpec((BM, H1), lambda i: (i, 0)),
            pl.BlockSpec((BM, H1), lambda i: (i, 0)),
            pl.BlockSpec((H1, H2), lambda i: (0, 0)),
            pl.BlockSpec((1, H2), lambda i: (0, 0)),
        ],
        out_specs=pl.BlockSpec((BM, H2), lambda i: (i, 0)),
        out_shape=jax.ShapeDtypeStruct((N, H2), jnp.float32),
    )(h, agg, W, b)


def kernel(edge_index, features, W1, b1, W2, b2):
    src = edge_index[0]
    dst = edge_index[1]
    agg1, llds, lsrc, lcnt = _agg1(src, dst, features)
    h = _dense1(features, agg1[:N], W1, b1.reshape(1, H1))
    agg2 = _agg2(llds, lsrc, lcnt, h)
    return _dense2(h, agg2[:N], W2, b2.reshape(1, H2))


# speedup vs baseline: 6.4078x; 1.1579x over previous
import functools

import jax
import jax.numpy as jnp
from jax import lax
from jax.experimental import pallas as pl
from jax.experimental.pallas import tpu as pltpu
from jax.experimental.pallas import tpu_sc as plsc

N = 10000
E = 320000
D = 128
H1 = 16
H2 = 7

NC = 2
NS = 16
NW = NC * NS
NPW = 320
NPAD = NW * NPW

CH = 6400
NCHUNK = E // CH
G = 64
LMAX = E + NCHUNK * G

_SC_PARAMS = dict(
    needs_layout_passes=False,
)


def _mesh():
    return plsc.VectorSubcoreMesh(core_axis_name="c", subcore_axis_name="s")


def _agg1_build():
    FD = D
    NV = FD // 16

    @functools.partial(
        pl.kernel,
        out_type=(
            jax.ShapeDtypeStruct((NPAD, FD), jnp.float32),
            jax.ShapeDtypeStruct((NW * LMAX,), jnp.int32),
            jax.ShapeDtypeStruct((NW * LMAX,), jnp.int32),
            jax.ShapeDtypeStruct((NW * 16,), jnp.int32),
        ),
        mesh=_mesh(),
        compiler_params=pltpu.CompilerParams(**_SC_PARAMS),
        scratch_types=[
            pltpu.VMEM((CH,), jnp.int32),
            pltpu.VMEM((CH,), jnp.int32),
            pltpu.VMEM((CH,), jnp.int32),
            pltpu.VMEM((CH,), jnp.int32),
            pltpu.VMEM((CH + 16,), jnp.int32),
            pltpu.VMEM((CH + 16,), jnp.int32),
            pltpu.VMEM((2, G, FD), jnp.float32),
            pltpu.VMEM((NPW + 1, FD), jnp.float32),
            pltpu.VMEM((16,), jnp.int32),
            pltpu.SemaphoreType.DMA,
            pltpu.SemaphoreType.DMA,
            pltpu.SemaphoreType.DMA,
            pltpu.SemaphoreType.DMA,
            pltpu.SemaphoreType.DMA,
        ],
    )
    def agg(src_hbm, dst_hbm, feat_hbm,
            out_hbm, llds_hbm, lsrc_hbm, lcnt_hbm,
            dst_v0, dst_v1, src_v0, src_v1, cld_v, csrc_v, rows_v, acc_v, cnt_v,
            sem0, sem1, sem_st, sem_e0, sem_e1):
        dstbanks = [dst_v0, dst_v1]
        srcbanks = [src_v0, src_v1]
        wid = lax.axis_index("s") * NC + lax.axis_index("c")
        lo = wid * NPW

        neg_inf = jnp.full((16,), -jnp.inf, dtype=jnp.float32)

        def init_acc(r, _):
            for j in range(NV):
                acc_v[r, pl.ds(j * 16, 16)] = neg_inf
            return 0
        lax.fori_loop(0, NPW + 1, init_acc, 0)

        lane = lax.iota(jnp.int32, 16)
        sems = [sem0, sem1]

        def fire_gather(b, buf, total):
            bbase = pl.multiple_of(b * G, 8)
            pltpu.async_copy(
                feat_hbm.at[csrc_v.at[pl.ds(bbase, G)]],
                rows_v.at[buf], sems[buf])
            pltpu.async_copy(
                cld_v.at[pl.ds(bbase, G)],
                llds_hbm.at[pl.ds(pl.multiple_of(wid * LMAX + total + bbase, 8), G)], sem_st)
            pltpu.async_copy(
                csrc_v.at[pl.ds(bbase, G)],
                lsrc_hbm.at[pl.ds(pl.multiple_of(wid * LMAX + total + bbase, 8), G)], sem_st)

        def wait_gather(buf):
            pltpu.make_async_copy(
                feat_hbm.at[csrc_v.at[pl.ds(0, G)]], rows_v.at[buf], sems[buf]
            ).wait()

        esems = [sem_e0, sem_e1]

        def fire_edges(c, bank):
            ebase = pl.multiple_of(c * CH, 8)
            pltpu.async_copy(dst_hbm.at[pl.ds(ebase, CH)], dstbanks[bank], esems[bank])
            pltpu.async_copy(src_hbm.at[pl.ds(ebase, CH)], srcbanks[bank], esems[bank])

        def wait_edges(bank):
            pltpu.make_async_copy(
                dst_hbm.at[pl.ds(0, CH)], dstbanks[bank], esems[bank]).wait()
            pltpu.make_async_copy(
                src_hbm.at[pl.ds(0, CH)], srcbanks[bank], esems[bank]).wait()

        def chunk_body(c, bank, total):
            wait_edges(bank)
            dstb = dstbanks[bank]
            srcb = srcbanks[bank]

            @plsc.parallel_loop(0, CH // 16, unroll=4,
                                carry=jnp.zeros((16,), jnp.int32))
            def filt(g, cnt):
                d16 = dstb[pl.ds(g * 16, 16)]
                s16 = srcb[pl.ds(g * 16, 16)]
                m = (d16 >= lo) & (d16 < lo + NPW)
                pos = cnt + plsc.cumsum(m.astype(jnp.int32)) - 1
                plsc.store_scatter(cld_v, [pos], d16 - lo, mask=m)
                plsc.store_scatter(csrc_v, [pos], s16, mask=m)
                return cnt + plsc.all_reduce_population_count(m)
            cnt_vec = filt
            cnt = jnp.max(cnt_vec)
            nb = (cnt + G - 1) >> 6

            def sanit(g, _):
                p16 = g * 16 + lane
                mv = p16 < cnt
                old_ld = cld_v[pl.ds(g * 16, 16)]
                old_sr = csrc_v[pl.ds(g * 16, 16)]
                cld_v[pl.ds(g * 16, 16)] = jnp.where(mv, old_ld, NPW)
                csrc_v[pl.ds(g * 16, 16)] = jnp.where(mv, old_sr, (p16 * 31) & 8191)
                return 0
            lax.fori_loop(cnt >> 4, (nb * G) >> 4, sanit, 0)

            @pl.when(nb > 0)
            def _():
                fire_gather(0, 0, total)

            def do_batch(b, buf):
                wait_gather(buf)

                def group_body(g, _):
                    ld_vec = cld_v[pl.ds(b * G + g * 16, 16)]
                    for j in range(16):
                        ld = ld_vec[j]
                        e = g * 16 + j

                        @plsc.parallel_loop(0, NV, unroll=NV)
                        def _(k):
                            sl = pl.ds(k * 16, 16)
                            r = rows_v[buf, e, sl]
                            acc_v[ld, sl] = jnp.maximum(acc_v[ld, sl], r)
                    return 0
                lax.fori_loop(0, G // 16, group_body, 0)

            def pair_body(p, _):
                b0 = 2 * p
                b1 = b0 + 1

                @pl.when(b1 < nb)
                def _():
                    fire_gather(b1, 1, total)
                do_batch(b0, 0)

                @pl.when(b1 < nb)
                def _():
                    @pl.when(b1 + 1 < nb)
                    def _():
                        fire_gather(b1 + 1, 0, total)
                    do_batch(b1, 1)
                return 0
            lax.fori_loop(0, (nb + 1) >> 1, pair_body, 0)

            def drain(b, _):
                bb = pl.multiple_of(b * G, 8)
                pltpu.make_async_copy(
                    cld_v.at[pl.ds(bb, G)],
                    llds_hbm.at[pl.ds(pl.multiple_of(wid * LMAX + total + bb, 8), G)], sem_st).wait()
                pltpu.make_async_copy(
                    csrc_v.at[pl.ds(bb, G)],
                    lsrc_hbm.at[pl.ds(pl.multiple_of(wid * LMAX + total + bb, 8), G)], sem_st).wait()
                return 0
            lax.fori_loop(0, nb, drain, 0)
            return total + nb * G

        fire_edges(0, 0)

        def cpair(pc, total):
            c0 = 2 * pc
            fire_edges(c0 + 1, 1)
            total = chunk_body(c0, 0, total)

            @pl.when(c0 + 2 < NCHUNK)
            def _():
                fire_edges(c0 + 2, 0)
            total = chunk_body(c0 + 1, 1, total)
            return total
        total = lax.fori_loop(0, NCHUNK // 2, cpair, 0)

        cnt_v[pl.ds(0, 16)] = jnp.zeros((16,), jnp.int32) + total
        pltpu.sync_copy(cnt_v, lcnt_hbm.at[pl.ds(pl.multiple_of(wid * 16, 8), 16)])

        pltpu.sync_copy(acc_v.at[pl.ds(0, NPW)], out_hbm.at[pl.ds(lo, NPW)])

    return agg


def _agg2_build():
    FD = H1
    SP = 1024
    NBSP = SP // G

    @functools.partial(
        pl.kernel,
        out_type=jax.ShapeDtypeStruct((NPAD, FD), jnp.float32),
        mesh=_mesh(),
        compiler_params=pltpu.CompilerParams(
            use_tc_tiling_on_sc=False, **_SC_PARAMS),
        scratch_types=[
            pltpu.VMEM((SP,), jnp.int32),
            pltpu.VMEM((SP,), jnp.int32),
            pltpu.VMEM((2, G, FD), jnp.float32),
            pltpu.VMEM((NPW + 1, FD), jnp.float32),
            pltpu.VMEM((NPW + 1, FD), jnp.float32),
            pltpu.VMEM((16,), jnp.int32),
            pltpu.SemaphoreType.DMA,
            pltpu.SemaphoreType.DMA,
            pltpu.SemaphoreType.DMA,
        ],
    )
    def agg(llds_hbm, lsrc_hbm, lcnt_hbm, feat_hbm, out_hbm,
            lds_v, src_v, rows_v, acc_a, acc_b, cnt_v, sem0, sem1, sem_sp):
        wid = lax.axis_index("s") * NC + lax.axis_index("c")
        lo = wid * NPW

        neg_inf = jnp.full((16,), -jnp.inf, dtype=jnp.float32)

        @plsc.parallel_loop(0, NPW + 1, unroll=4)
        def _(r):
            acc_a[r, pl.ds(0, 16)] = neg_inf
            acc_b[r, pl.ds(0, 16)] = neg_inf

        pltpu.sync_copy(lcnt_hbm.at[pl.ds(pl.multiple_of(wid * 16, 8), 16)], cnt_v)
        total = cnt_v[pl.ds(0, 16)][0]
        nsp = (total + SP - 1) >> 10

        sems = [sem0, sem1]

        def span_load(s):
            sbase = pl.multiple_of(s * SP, 8)
            sz = jnp.minimum(total - s * SP, SP)
            del sz
            pltpu.sync_copy(llds_hbm.at[pl.ds(pl.multiple_of(wid * LMAX + sbase, 8), SP)], lds_v)
            pltpu.sync_copy(lsrc_hbm.at[pl.ds(pl.multiple_of(wid * LMAX + sbase, 8), SP)], src_v)

        def fire_gather(b, buf):
            bbase = pl.multiple_of(b * G, 8)
            pltpu.async_copy(
                feat_hbm.at[src_v.at[pl.ds(bbase, G)]], rows_v.at[buf], sems[buf])

        def wait_gather(buf):
            pltpu.make_async_copy(
                feat_hbm.at[src_v.at[pl.ds(0, G)]], rows_v.at[buf], sems[buf]
            ).wait()

        def span_body(s, _):
            span_load(s)
            nb = jnp.minimum(total - s * SP, SP) >> 6

            @pl.when(nb > 0)
            def _():
                fire_gather(0, 0)

            def do_batch(b, buf):
                wait_gather(buf)

                def group_body(g, _):
                    ld_vec = lds_v[pl.ds(b * G + g * 16, 16)]
                    accs2 = [acc_a, acc_b]
                    for j in range(16):
                        ld = ld_vec[j]
                        e = g * 16 + j
                        acc = accs2[j & 1]
                        r = rows_v[buf, e, pl.ds(0, 16)]
                        acc[ld, pl.ds(0, 16)] = jnp.maximum(
                            acc[ld, pl.ds(0, 16)], r)
                    return 0
                lax.fori_loop(0, G // 16, group_body, 0)

            def pair_body(p, _):
                b0 = 2 * p
                b1 = b0 + 1

                @pl.when(b1 < nb)
                def _():
                    fire_gather(b1, 1)
                do_batch(b0, 0)

                @pl.when(b1 < nb)
                def _():
                    @pl.when(b1 + 1 < nb)
                    def _():
                        fire_gather(b1 + 1, 0)
                    do_batch(b1, 1)
                return 0
            lax.fori_loop(0, (nb + 1) >> 1, pair_body, 0)
            return 0
        lax.fori_loop(0, nsp, span_body, 0)

        @plsc.parallel_loop(0, NPW, unroll=4)
        def _(r):
            acc_a[r, pl.ds(0, 16)] = jnp.maximum(
                acc_a[r, pl.ds(0, 16)], acc_b[r, pl.ds(0, 16)])

        pltpu.sync_copy(acc_a.at[pl.ds(0, NPW)], out_hbm.at[pl.ds(lo, NPW)])

    return agg


_agg1 = _agg1_build()
_agg2 = _agg2_build()


def _dense1(x, agg3, W, b):
    BM = 1000

    def body(x_ref, a_ref, w_ref, b_ref, o_ref):
        a = a_ref[...]
        a = jnp.where(a == -jnp.inf, 0.0, a)
        r = x_ref[...] + a
        h = jnp.dot(r, w_ref[...], preferred_element_type=jnp.float32) + b_ref[...]
        o_ref[...] = jnp.maximum(h, 0.0)

    return pl.pallas_call(
        body,
        grid=(N // BM,),
        in_specs=[
            pl.BlockSpec((BM, D), lambda i: (i, 0)),
            pl.BlockSpec((BM, D), lambda i: (i, 0)),
            pl.BlockSpec((D, H1), lambda i: (0, 0)),
            pl.BlockSpec((1, H1), lambda i: (0, 0)),
        ],
        out_specs=pl.BlockSpec((BM, H1), lambda i: (i, 0)),
        out_shape=jax.ShapeDtypeStruct((N, H1), jnp.float32),
    )(x, agg3, W, b)


def _dense2(h, agg, W, b):
    BM = 1000

    def body(h_ref, a_ref, w_ref, b_ref, o_ref):
        a = a_ref[...]
        a = jnp.where(a == -jnp.inf, 0.0, a)
        r = h_ref[...] + a
        y = jnp.dot(r, w_ref[...], preferred_element_type=jnp.float32) + b_ref[...]
        m = jnp.max(y, axis=-1, keepdims=True)
        z = y - m
        o_ref[...] = z - jnp.log(jnp.sum(jnp.exp(z), axis=-1, keepdims=True))

    return pl.pallas_call(
        body,
        grid=(N // BM,),
        in_specs=[
            pl.BlockSpec((BM, H1), lambda i: (i, 0)),
            pl.BlockSpec((BM, H1), lambda i: (i, 0)),
            pl.BlockSpec((H1, H2), lambda i: (0, 0)),
            pl.BlockSpec((1, H2), lambda i: (0, 0)),
        ],
        out_specs=pl.BlockSpec((BM, H2), lambda i: (i, 0)),
        out_shape=jax.ShapeDtypeStruct((N, H2), jnp.float32),
    )(h, agg, W, b)


def kernel(edge_index, features, W1, b1, W2, b2):
    src = edge_index[0]
    dst = edge_index[1]
    agg1, llds, lsrc, lcnt = _agg1(src, dst, features)
    h = _dense1(features, agg1[:N], W1, b1.reshape(1, H1))
    agg2 = _agg2(llds, lsrc, lcnt, h)
    return _dense2(h, agg2[:N], W2, b2.reshape(1, H2))
